# Initial kernel scaffold; baseline (speedup 1.0000x reference)
import jax
import jax.numpy as jnp
from jax.experimental import pallas as pl


def kernel(cat, table):
    raise NotImplementedError("write your pallas kernel here")


# speedup vs baseline: 3.6513x; 3.6513x over previous
import functools

import jax
import jax.numpy as jnp
from jax import lax
from jax.experimental import pallas as pl
from jax.experimental.pallas import tpu as pltpu
from jax.experimental.pallas import tpu_sc as plsc

_BATCH = 4096
_N_FIELDS = 26
_PER_FIELD_VOCAB = 50
_EMBED_DIM = 64

_NC = 2
_NS = 16
_NW = _NC * _NS

_B_FLAT = _BATCH * _N_FIELDS
_PER_W = _B_FLAT // _NW
_STEP = 128
_N_STEPS = _PER_W // _STEP
_LANES = 16


def _body(cat_hbm, table_hbm, out_hbm, cat_v, idx_v, rows0, rows1, sem0, sem1):
    wid = lax.axis_index("s") * _NC + lax.axis_index("c")

    pltpu.sync_copy(cat_hbm.at[pl.ds(wid * _PER_W, _PER_W)], cat_v)

    lane = lax.broadcasted_iota(jnp.int32, (_LANES,), 0)
    for j in range(_N_STEPS):
        for k in range(_STEP // _LANES):
            p = j * _STEP + k * _LANES
            off = ((p + lane) % _N_FIELDS) * _PER_FIELD_VOCAB
            idx_v[j, pl.ds(k * _LANES, _LANES)] = cat_v[pl.ds(p, _LANES)] + off

    base = wid * _PER_W

    def gather(j, buf, sem):
        return pltpu.async_copy(table_hbm.at[idx_v.at[j]], buf, sem)

    bufs = (rows0, rows1)
    sems = (sem0, sem1)
    prev = gather(0, bufs[0], sems[0])
    for j in range(1, _N_STEPS):
        cur = gather(j, bufs[j % 2], sems[j % 2])
        prev.wait()
        pltpu.sync_copy(
            bufs[(j - 1) % 2], out_hbm.at[pl.ds(base + (j - 1) * _STEP, _STEP)]
        )
        prev = cur
    prev.wait()
    pltpu.sync_copy(
        bufs[(_N_STEPS - 1) % 2],
        out_hbm.at[pl.ds(base + (_N_STEPS - 1) * _STEP, _STEP)],
    )


@functools.partial(jax.jit, static_argnames=())
def _lookup(cat2d, table):
    mesh = plsc.VectorSubcoreMesh(
        core_axis_name="c", subcore_axis_name="s", num_cores=_NC, num_subcores=_NS
    )
    k = pl.kernel(
        _body,
        out_type=jax.ShapeDtypeStruct((_B_FLAT, _EMBED_DIM), jnp.float32),
        mesh=mesh,
        scratch_types=[
            pltpu.VMEM((_PER_W,), jnp.int32),
            pltpu.VMEM((_N_STEPS, _STEP), jnp.int32),
            pltpu.VMEM((_STEP, _EMBED_DIM), jnp.float32),
            pltpu.VMEM((_STEP, _EMBED_DIM), jnp.float32),
            pltpu.SemaphoreType.DMA,
            pltpu.SemaphoreType.DMA,
        ],
        compiler_params=pltpu.CompilerParams(use_tc_tiling_on_sc=False),
    )
    return k(cat2d, table)


def kernel(cat, table):
    out = _lookup(cat.reshape(_B_FLAT), table)
    return out.reshape(_BATCH, _N_FIELDS, _EMBED_DIM)


# speedup vs baseline: 3.6696x; 1.0050x over previous
import functools

import jax
import jax.numpy as jnp
from jax import lax
from jax.experimental import pallas as pl
from jax.experimental.pallas import tpu as pltpu
from jax.experimental.pallas import tpu_sc as plsc

_BATCH = 4096
_N_FIELDS = 26
_PER_FIELD_VOCAB = 50
_EMBED_DIM = 64

_NC = 2
_NS = 16
_NW = _NC * _NS

_B_FLAT = _BATCH * _N_FIELDS
_PER_W = _B_FLAT // _NW
_STEP = 128
_N_STEPS = _PER_W // _STEP
_LANES = 16


_MACRO = 2 * _STEP
_N_MACRO = _PER_W // _MACRO
_NBUF = 4
_LAG = 2


def _body(cat_hbm, table_hbm, out_hbm, cat_v, idx_v, bufs, gsems, wsems):
    wid = lax.axis_index("s") * _NC + lax.axis_index("c")

    pltpu.sync_copy(cat_hbm.at[pl.ds(wid * _PER_W, _PER_W)], cat_v)

    lane = lax.broadcasted_iota(jnp.int32, (_LANES,), 0)
    for j in range(_N_STEPS):
        for k in range(_STEP // _LANES):
            p = j * _STEP + k * _LANES
            off = ((p + lane) % _N_FIELDS) * _PER_FIELD_VOCAB
            idx_v[j, pl.ds(k * _LANES, _LANES)] = cat_v[pl.ds(p, _LANES)] + off

    base = wid * _PER_W
    pend_g = [None] * _NBUF
    pend_w = [None] * _NBUF
    for m in range(_N_MACRO + _LAG):
        if m < _N_MACRO:
            s = m % _NBUF
            if pend_w[s] is not None:
                pend_w[s].wait()
            c0 = pltpu.async_copy(
                table_hbm.at[idx_v.at[2 * m]], bufs[s].at[pl.ds(0, _STEP)], gsems[s]
            )
            c1 = pltpu.async_copy(
                table_hbm.at[idx_v.at[2 * m + 1]],
                bufs[s].at[pl.ds(_STEP, _STEP)],
                gsems[s],
            )
            pend_g[s] = (c0, c1)
        i = m - _LAG
        if 0 <= i < _N_MACRO:
            s = i % _NBUF
            c0, c1 = pend_g[s]
            c0.wait()
            c1.wait()
            pend_w[s] = pltpu.async_copy(
                bufs[s], out_hbm.at[pl.ds(base + i * _MACRO, _MACRO)], wsems[s]
            )
    for s in range(_NBUF):
        if pend_w[s] is not None:
            pend_w[s].wait()


@functools.partial(jax.jit, static_argnames=())
def _lookup(cat2d, table):
    mesh = plsc.VectorSubcoreMesh(
        core_axis_name="c", subcore_axis_name="s", num_cores=_NC, num_subcores=_NS
    )
    k = pl.kernel(
        _body,
        out_type=jax.ShapeDtypeStruct((_B_FLAT, _EMBED_DIM), jnp.float32),
        mesh=mesh,
        scratch_types=[
            pltpu.VMEM((_PER_W,), jnp.int32),
            pltpu.VMEM((_N_STEPS, _STEP), jnp.int32),
            [pltpu.VMEM((_MACRO, _EMBED_DIM), jnp.float32) for _ in range(_NBUF)],
            [pltpu.SemaphoreType.DMA for _ in range(_NBUF)],
            [pltpu.SemaphoreType.DMA for _ in range(_NBUF)],
        ],
        compiler_params=pltpu.CompilerParams(use_tc_tiling_on_sc=False),
    )
    return k(cat2d, table)


def kernel(cat, table):
    out = _lookup(cat.reshape(_B_FLAT), table)
    return out.reshape(_BATCH, _N_FIELDS, _EMBED_DIM)
